# baseline (speedup 1.0000x reference)
import functools

import jax
import jax.numpy as jnp
from jax import lax
from jax.experimental import pallas as pl
from jax.experimental.pallas import tpu as pltpu
from jax.experimental.pallas import tpu_sc as plsc

BATCH = 16384
DIM = 32
V = 100000
CHUNK = 128
TBLK = 4096
VA = (V // CHUNK) * CHUNK
STEPS = [(i * TBLK, TBLK) for i in range(VA // TBLK)] + [
    ((VA // TBLK) * TBLK, VA - (VA // TBLK) * TBLK)]
NQ = 4


def _tc_transpose4(gu_t, mu_t, gi_t, mi_t, tails):
    nstep = len(STEPS)

    def body(gu, mu, gi, mi, tl, out, inb, outb, tlb, insem, outsem, tsem):
        tabs = (gu, mu, gi, mi)
        eye = jnp.eye(DIM, dtype=jnp.float32)
        dn = (((0,), (0,)), ((), ()))

        def in_copy(i, t):
            off, w = STEPS[i]
            return pltpu.make_async_copy(
                tabs[t].at[:, pl.ds(off, w)],
                inb.at[i % 2, t, slice(None), pl.ds(0, w)],
                insem.at[i % 2, t])

        def out_copy(i):
            off, w = STEPS[i] if i < nstep else (VA, DIM)
            return pltpu.make_async_copy(
                outb.at[i % NQ, pl.ds(0, w)],
                out.at[pl.ds(off, w)],
                outsem.at[i % NQ])

        tcp = pltpu.make_async_copy(tl, tlb, tsem)
        tcp.start()
        for t in range(4):
            in_copy(0, t).start()
        for i in range(nstep):
            if i + 1 < nstep:
                for t in range(4):
                    in_copy(i + 1, t).start()
            for t in range(4):
                in_copy(i, t).wait()
            if i >= NQ:
                out_copy(i - NQ).wait()
            w = STEPS[i][1]
            pieces = [
                jax.lax.dot_general(inb[i % 2, t, :, pl.ds(0, w)], eye, dn,
                                    preferred_element_type=jnp.float32)
                for t in range(4)
            ]
            outb[i % NQ, pl.ds(0, w)] = jnp.concatenate(pieces, axis=1)
            out_copy(i).start()
        tcp.wait()
        out_copy(nstep - NQ + 1).wait()
        tpieces = [
            jax.lax.dot_general(tlb[pl.ds(t * DIM, DIM)], eye, dn,
                                preferred_element_type=jnp.float32)
            for t in range(4)
        ]
        outb[(nstep + 1) % NQ, pl.ds(0, DIM)] = jnp.concatenate(
            tpieces, axis=1)
        out_copy(nstep + 1).start()
        for i in (nstep - NQ, nstep - 2, nstep - 1, nstep + 1):
            out_copy(i).wait()

    anyspec = pl.BlockSpec(memory_space=pl.ANY)
    return pl.pallas_call(
        body,
        in_specs=[anyspec] * 5,
        out_specs=anyspec,
        out_shape=jax.ShapeDtypeStruct((V, 4 * DIM), jnp.float32),
        scratch_shapes=[
            pltpu.VMEM((2, 4, DIM, TBLK), jnp.float32),
            pltpu.VMEM((NQ, TBLK, 4 * DIM), jnp.float32),
            pltpu.VMEM((4 * DIM, DIM), jnp.float32),
            pltpu.SemaphoreType.DMA((2, 4)),
            pltpu.SemaphoreType.DMA((NQ,)),
            pltpu.SemaphoreType.DMA,
        ],
    )(gu_t, mu_t, gi_t, mi_t, tails)


def _sc_gather(user2d, item2d, t4):
    info = plsc.get_sparse_core_info()
    nc, ns = info.num_cores, info.num_subcores
    nw = nc * ns
    b_per_w = BATCH // nw
    n_chunks = b_per_w // CHUNK
    mesh = plsc.VectorSubcoreMesh(core_axis_name="c", subcore_axis_name="s")
    out_sds = jax.ShapeDtypeStruct((BATCH, 4 * DIM), jnp.float32)

    @functools.partial(
        pl.kernel,
        mesh=mesh,
        out_type=out_sds,
        compiler_params=pltpu.CompilerParams(use_tc_tiling_on_sc=False),
        scratch_types=[
            pltpu.VMEM((n_chunks, CHUNK), jnp.int32),
            pltpu.VMEM((n_chunks, CHUNK), jnp.int32),
            pltpu.VMEM((2, CHUNK, 4 * DIM), jnp.float32),
            pltpu.SemaphoreType.DMA,
            pltpu.SemaphoreType.DMA,
        ],
    )
    def k(user_hbm, item_hbm, t4_hbm, out_x, idx_u, idx_i, stg, sem, osem):
        wid = lax.axis_index("s") * nc + lax.axis_index("c")
        pltpu.sync_copy(user_hbm.at[pl.ds(wid * n_chunks, n_chunks)], idx_u)
        pltpu.sync_copy(item_hbm.at[pl.ds(wid * n_chunks, n_chunks)], idx_i)
        for c in range(n_chunks):
            cu = pltpu.async_copy(t4_hbm.at[idx_u.at[c]], stg.at[0], sem)
            ci = pltpu.async_copy(t4_hbm.at[idx_i.at[c]], stg.at[1], sem)
            rows = pl.ds(wid * b_per_w + c * CHUNK, CHUNK)
            cu.wait()
            ou = pltpu.async_copy(
                stg.at[0, slice(None), pl.ds(0, 2 * DIM)],
                out_x.at[rows, pl.ds(0, 2 * DIM)], osem)
            ci.wait()
            oi = pltpu.async_copy(
                stg.at[1, slice(None), pl.ds(2 * DIM, 2 * DIM)],
                out_x.at[rows, pl.ds(2 * DIM, 2 * DIM)], osem)
            ou.wait()
            oi.wait()

    return k(user2d, item2d, t4)


def _tc_mlp(x, w1u_t, w1i_t, w2t, wfg, wfh, b1r, b2r, bfr):
    bm = 2048
    grid = (BATCH // bm,)

    def body(x_ref, w1u_ref, w1i_ref, w2_ref, wfg_ref, wfh_ref,
             b1_ref, b2_ref, bf_ref, out_ref):
        xb = x_ref[...]
        g = xb[:, 0:DIM] * xb[:, 2 * DIM:3 * DIM]
        h1 = jnp.maximum(
            jnp.dot(xb[:, DIM:2 * DIM], w1u_ref[...],
                    preferred_element_type=jnp.float32)
            + jnp.dot(xb[:, 3 * DIM:], w1i_ref[...],
                      preferred_element_type=jnp.float32)
            + b1_ref[...], 0.0)
        h2 = jnp.maximum(
            jnp.dot(h1, w2_ref[...], preferred_element_type=jnp.float32)
            + b2_ref[...], 0.0)
        out_ref[...] = (
            jnp.sum(g * wfg_ref[...], axis=1, keepdims=True)
            + jnp.sum(h2 * wfh_ref[...], axis=1, keepdims=True)
            + bf_ref[...])

    full = lambda shape: pl.BlockSpec(shape, lambda i: tuple(0 for _ in shape))
    return pl.pallas_call(
        body,
        grid=grid,
        in_specs=[
            pl.BlockSpec((bm, 4 * DIM), lambda i: (i, 0)),
            full((DIM, 64)), full((DIM, 64)), full((64, DIM)),
            full((1, DIM)), full((1, DIM)),
            full((1, 64)), full((1, DIM)), full((1, 1)),
        ],
        out_specs=pl.BlockSpec((bm, 1), lambda i: (i, 0)),
        out_shape=jax.ShapeDtypeStruct((BATCH, 1), jnp.float32),
    )(x, w1u_t, w1i_t, w2t, wfg, wfh, b1r, b2r, bfr)


def kernel(user, item, gmf_user_emb, gmf_item_emb, mlp_user_emb, mlp_item_emb,
           W1, b1, W2, b2, Wf, bf):
    user2d = user.astype(jnp.int32).reshape(-1, CHUNK)
    item2d = item.astype(jnp.int32).reshape(-1, CHUNK)
    tails = jnp.concatenate(
        [gmf_user_emb.T[:, VA:], mlp_user_emb.T[:, VA:],
         gmf_item_emb.T[:, VA:], mlp_item_emb.T[:, VA:]], axis=0)
    t4 = _tc_transpose4(gmf_user_emb.T, mlp_user_emb.T,
                        gmf_item_emb.T, mlp_item_emb.T, tails)
    x = _sc_gather(user2d, item2d, t4)
    out = _tc_mlp(x, W1[:, :DIM].T, W1[:, DIM:].T, W2.T,
                  Wf[:, :DIM], Wf[:, DIM:],
                  b1.reshape(1, 64), b2.reshape(1, DIM), bf.reshape(1, 1))
    return out[:, 0]


# speedup vs baseline: 1.0213x; 1.0213x over previous
import functools

import jax
import jax.numpy as jnp
from jax import lax
from jax.experimental import pallas as pl
from jax.experimental.pallas import tpu as pltpu
from jax.experimental.pallas import tpu_sc as plsc

BATCH = 16384
DIM = 32
V = 100000
CHUNK = 128
TBLK = 4096


def _tc_transpose4(gu_t, mu_t, gi_t, mi_t):
    grid = (pl.cdiv(V, TBLK),)

    def body(gu_ref, mu_ref, gi_ref, mi_ref, out_ref):
        eye = jnp.eye(DIM, dtype=jnp.float32)
        dn = (((0,), (0,)), ((), ()))

        def tr(ref):
            return jax.lax.dot_general(ref[...], eye, dn,
                                       preferred_element_type=jnp.float32)

        out_ref[...] = jnp.concatenate(
            [tr(gu_ref), tr(mu_ref), tr(gi_ref), tr(mi_ref)],
            axis=1)

    spec = pl.BlockSpec((DIM, TBLK), lambda i: (0, i))
    return pl.pallas_call(
        body,
        grid=grid,
        in_specs=[spec, spec, spec, spec],
        out_specs=pl.BlockSpec((TBLK, 4 * DIM), lambda i: (i, 0)),
        out_shape=jax.ShapeDtypeStruct((V, 4 * DIM), jnp.float32),
    )(gu_t, mu_t, gi_t, mi_t)


def _sc_gather(user2d, item2d, t4):
    info = plsc.get_sparse_core_info()
    nc, ns = info.num_cores, info.num_subcores
    nw = nc * ns
    b_per_w = BATCH // nw
    n_chunks = b_per_w // CHUNK
    mesh = plsc.VectorSubcoreMesh(core_axis_name="c", subcore_axis_name="s")
    out_sds = jax.ShapeDtypeStruct((BATCH, 4 * DIM), jnp.float32)

    @functools.partial(
        pl.kernel,
        mesh=mesh,
        out_type=out_sds,
        compiler_params=pltpu.CompilerParams(use_tc_tiling_on_sc=False),
        scratch_types=[
            pltpu.VMEM((n_chunks, CHUNK), jnp.int32),
            pltpu.VMEM((n_chunks, CHUNK), jnp.int32),
            pltpu.VMEM((2, CHUNK, 4 * DIM), jnp.float32),
            pltpu.SemaphoreType.DMA,
            pltpu.SemaphoreType.DMA,
        ],
    )
    def k(user_hbm, item_hbm, t4_hbm, out_x, idx_u, idx_i, stg, sem, osem):
        wid = lax.axis_index("s") * nc + lax.axis_index("c")
        pltpu.sync_copy(user_hbm.at[pl.ds(wid * n_chunks, n_chunks)], idx_u)
        pltpu.sync_copy(item_hbm.at[pl.ds(wid * n_chunks, n_chunks)], idx_i)
        for c in range(n_chunks):
            cu = pltpu.async_copy(t4_hbm.at[idx_u.at[c]], stg.at[0], sem)
            ci = pltpu.async_copy(t4_hbm.at[idx_i.at[c]], stg.at[1], sem)
            rows = pl.ds(wid * b_per_w + c * CHUNK, CHUNK)
            cu.wait()
            ou = pltpu.async_copy(
                stg.at[0, slice(None), pl.ds(0, 2 * DIM)],
                out_x.at[rows, pl.ds(0, 2 * DIM)], osem)
            ci.wait()
            oi = pltpu.async_copy(
                stg.at[1, slice(None), pl.ds(2 * DIM, 2 * DIM)],
                out_x.at[rows, pl.ds(2 * DIM, 2 * DIM)], osem)
            ou.wait()
            oi.wait()

    return k(user2d, item2d, t4)


def _tc_mlp(x, w1u_t, w1i_t, w2t, wfg, wfh, b1r, b2r, bfr):
    bm = 2048
    grid = (BATCH // bm,)

    def body(x_ref, w1u_ref, w1i_ref, w2_ref, wfg_ref, wfh_ref,
             b1_ref, b2_ref, bf_ref, out_ref):
        xb = x_ref[...]
        g = xb[:, 0:DIM] * xb[:, 2 * DIM:3 * DIM]
        h1 = jnp.maximum(
            jnp.dot(xb[:, DIM:2 * DIM], w1u_ref[...],
                    preferred_element_type=jnp.float32)
            + jnp.dot(xb[:, 3 * DIM:], w1i_ref[...],
                      preferred_element_type=jnp.float32)
            + b1_ref[...], 0.0)
        h2 = jnp.maximum(
            jnp.dot(h1, w2_ref[...], preferred_element_type=jnp.float32)
            + b2_ref[...], 0.0)
        out_ref[...] = (
            jnp.sum(g * wfg_ref[...], axis=1, keepdims=True)
            + jnp.sum(h2 * wfh_ref[...], axis=1, keepdims=True)
            + bf_ref[...])

    full = lambda shape: pl.BlockSpec(shape, lambda i: tuple(0 for _ in shape))
    return pl.pallas_call(
        body,
        grid=grid,
        in_specs=[
            pl.BlockSpec((bm, 4 * DIM), lambda i: (i, 0)),
            full((DIM, 64)), full((DIM, 64)), full((64, DIM)),
            full((1, DIM)), full((1, DIM)),
            full((1, 64)), full((1, DIM)), full((1, 1)),
        ],
        out_specs=pl.BlockSpec((bm, 1), lambda i: (i, 0)),
        out_shape=jax.ShapeDtypeStruct((BATCH, 1), jnp.float32),
    )(x, w1u_t, w1i_t, w2t, wfg, wfh, b1r, b2r, bfr)


def kernel(user, item, gmf_user_emb, gmf_item_emb, mlp_user_emb, mlp_item_emb,
           W1, b1, W2, b2, Wf, bf):
    user2d = user.astype(jnp.int32).reshape(-1, CHUNK)
    item2d = item.astype(jnp.int32).reshape(-1, CHUNK)
    t4 = _tc_transpose4(gmf_user_emb.T, mlp_user_emb.T,
                        gmf_item_emb.T, mlp_item_emb.T)
    x = _sc_gather(user2d, item2d, t4)
    out = _tc_mlp(x, W1[:, :DIM].T, W1[:, DIM:].T, W2.T,
                  Wf[:, :DIM], Wf[:, DIM:],
                  b1.reshape(1, 64), b2.reshape(1, DIM), bf.reshape(1, 1))
    return out[:, 0]
